# baseline (speedup 1.0000x reference)
import jax
import jax.numpy as jnp
from jax import lax
from jax.experimental import pallas as pl
from jax.experimental.pallas import tpu as pltpu
from jax.experimental.pallas import tpu_sc as plsc

NC, NS, L = 2, 16, 16
NW = NC * NS
B = 16384
D = 32
NCTX = 5
V = 1000000
V4 = V // 4
BPW = B // NW
JPW = BPW * NCTX
SUB = 8
BSUB = BPW // SUB
JSUB = JPW // SUB
RCH = 64
WE_CH = BSUB // RCH
CE_CH = JSUB // RCH
GSUB = BSUB // L


def _w2v_body(tgt_hbm, ctx_hbm, wt_hbm, wc_hbm, out_hbm,
              tgt_idx, ctx_idx, tgt_rows, ctx_rows,
              we_a, we_b, ce_a, ce_b, out_v, sem):
    wid = lax.axis_index("s") * NC + lax.axis_index("c")

    pltpu.sync_copy(tgt_hbm.at[pl.ds(wid * BPW, BPW)], tgt_idx)
    pltpu.sync_copy(ctx_hbm.at[pl.ds(wid * JPW, JPW)], ctx_idx)

    @pl.loop(0, BPW // RCH)
    def _trows(r):
        for k in range(RCH // L):
            v = tgt_idx[pl.ds(r * RCH + k * L, L)]
            tgt_rows[r, pl.ds(k * L, L)] = lax.shift_right_logical(v, 2)

    @pl.loop(0, JPW // RCH)
    def _crows(r):
        for k in range(RCH // L):
            v = ctx_idx[pl.ds(r * RCH + k * L, L)]
            ctx_rows[r, pl.ds(k * L, L)] = lax.shift_right_logical(v, 2)

    we_bufs = [we_a, we_b]
    ce_bufs = [ce_a, ce_b]

    def fire(s):
        p = s % 2
        cps = []
        for c in range(WE_CH):
            cps.append(pltpu.async_copy(
                wt_hbm.at[tgt_rows.at[s * WE_CH + c]],
                we_bufs[p].at[pl.ds(c * RCH, RCH)], sem))
        for c in range(CE_CH):
            cps.append(pltpu.async_copy(
                wc_hbm.at[ctx_rows.at[s * CE_CH + c]],
                ce_bufs[p].at[pl.ds(c * RCH, RCH)], sem))
        return cps

    iota = lax.iota(jnp.int32, L)

    def compute(s):
        p = s % 2
        we4 = we_bufs[p]
        ce4 = ce_bufs[p]

        @pl.loop(0, GSUB)
        def _group(g):
            gbase = g * L
            b_sub = iota + gbase
            tidx = tgt_idx[pl.ds(s * BSUB + gbase, L)]
            wcol = lax.shift_left(jnp.bitwise_and(tidx, 3), 5)
            wecols = [
                plsc.load_gather(we4, [b_sub, wcol + d]) for d in range(D)
            ]
            for n in range(NCTX):
                j_sub = iota * NCTX + (gbase * NCTX + n)
                j_glob = j_sub + s * JSUB
                cidx = plsc.load_gather(ctx_idx, [j_glob])
                ccol = lax.shift_left(jnp.bitwise_and(cidx, 3), 5)
                acc = wecols[0] * plsc.load_gather(ce4, [j_sub, ccol])
                for d in range(1, D):
                    acc = acc + wecols[d] * plsc.load_gather(
                        ce4, [j_sub, ccol + d])
                plsc.store_scatter(out_v, [j_glob], acc)

    cps = fire(0)
    for s in range(SUB):
        for cp in cps:
            cp.wait()
        if s + 1 < SUB:
            nxt = fire(s + 1)
        compute(s)
        if s + 1 < SUB:
            cps = nxt

    pltpu.sync_copy(out_v, out_hbm.at[pl.ds(wid * JPW, JPW)])


@jax.jit
def kernel(target, context, W_target, W_context):
    tgt_flat = target.reshape(B)
    ctx_flat = context.reshape(B * NCTX)
    wt4 = W_target.reshape(V4, 4 * D)
    wc4 = W_context.reshape(V4, 4 * D)

    mesh = plsc.VectorSubcoreMesh(
        core_axis_name="c", subcore_axis_name="s",
        num_cores=NC, num_subcores=NS)
    out_flat = pl.kernel(
        _w2v_body,
        out_type=jax.ShapeDtypeStruct((B * NCTX,), jnp.float32),
        mesh=mesh,
        compiler_params=pltpu.CompilerParams(needs_layout_passes=False),
        scratch_types=[
            pltpu.VMEM((BPW,), jnp.int32),
            pltpu.VMEM((JPW,), jnp.int32),
            pltpu.VMEM((BPW // RCH, RCH), jnp.int32),
            pltpu.VMEM((JPW // RCH, RCH), jnp.int32),
            pltpu.VMEM((BSUB, 4 * D), jnp.float32),
            pltpu.VMEM((BSUB, 4 * D), jnp.float32),
            pltpu.VMEM((JSUB, 4 * D), jnp.float32),
            pltpu.VMEM((JSUB, 4 * D), jnp.float32),
            pltpu.VMEM((JPW,), jnp.float32),
            pltpu.SemaphoreType.DMA,
        ],
    )(tgt_flat, ctx_flat, wt4, wc4)
    return out_flat.reshape(B, NCTX)
